# baseline (speedup 1.0000x reference)
import functools

import jax
import jax.numpy as jnp
from jax import lax
from jax.experimental import pallas as pl
from jax.experimental.pallas import tpu as pltpu
from jax.experimental.pallas import tpu_sc as plsc

N = 10000
E = 320000
G = 20
NUM_KEYS = 500
DIN = 16
D = 128
H = 4
DH = 32
L = 3

NPAD = 10240
EF = E + N


def _embed_body(x_ref, w_ref, b_ref, pos_ref, h_ref):
    h = jnp.maximum(jnp.dot(x_ref[...], w_ref[...],
                            preferred_element_type=jnp.float32)
                    + b_ref[...][None, :], 0.0)
    h_ref[...] = h + pos_ref[...]


def _embed(x, w, b, pos_full):
    return pl.pallas_call(
        _embed_body,
        out_shape=jax.ShapeDtypeStruct((N, D), jnp.float32),
    )(x, w, b, pos_full)


def _proj_body(h_ref, wl_ref, wr_ref, xl_ref, xr_ref):
    h = h_ref[...]
    xl_ref[:N, :] = jnp.dot(h, wl_ref[...], preferred_element_type=jnp.float32)
    xl_ref[N:, :] = jnp.zeros((NPAD - N, D), jnp.float32)
    xr_ref[:N, :] = jnp.dot(h, wr_ref[...], preferred_element_type=jnp.float32)
    xr_ref[N:, :] = jnp.zeros((NPAD - N, D), jnp.float32)


def _proj(h, wl, wr):
    return pl.pallas_call(
        _proj_body,
        out_shape=(jax.ShapeDtypeStruct((NPAD, D), jnp.float32),
                   jax.ShapeDtypeStruct((NPAD, D), jnp.float32)),
    )(h, wl, wr)


def _combine_body(num_ref, den_ref, b_ref, h_ref, lng_ref, lnb_ref, out_ref):
    num = num_ref[0, :N, :] + num_ref[1, :N, :]
    den = den_ref[0, :N, :H] + den_ref[1, :N, :H]
    den = jnp.maximum(den, 1e-16)
    r = 1.0 / den
    hsel = (lax.broadcasted_iota(jnp.int32, (H, D), 1) // DH
            == lax.broadcasted_iota(jnp.int32, (H, D), 0)).astype(jnp.float32)
    r128 = jnp.dot(r, hsel, preferred_element_type=jnp.float32)
    msg = jnp.maximum(num * r128 + b_ref[...][None, :], 0.0)
    hh = h_ref[...] + msg
    mu = jnp.mean(hh, axis=-1, keepdims=True)
    var = jnp.mean((hh - mu) ** 2, axis=-1, keepdims=True)
    out_ref[...] = (lng_ref[...][None, :] * (hh - mu)
                    / jnp.sqrt(var + 1e-5) + lnb_ref[...][None, :])


def _combine(num, den, bias, h, ln_g, ln_b):
    return pl.pallas_call(
        _combine_body,
        out_shape=jax.ShapeDtypeStruct((N, D), jnp.float32),
    )(num, den, bias, h, ln_g, ln_b)


def _pool_heads_body(h_ref, wg1_ref, bg1_ref, wg2_ref, bg2_ref,
                     wp1_ref, bp1_ref, lng_ref, lnb_ref, wp2_ref, bp2_ref,
                     wv1_ref, bv1_ref, wv2_ref, bv2_ref,
                     pol_ref, val_ref):
    h = h_ref[...]
    gate_hid = jnp.maximum(jnp.dot(h, wg1_ref[...],
                                   preferred_element_type=jnp.float32)
                           + bg1_ref[...][None, :], 0.0)
    gate = jnp.dot(gate_hid, wg2_ref[...],
                   preferred_element_type=jnp.float32) + bg2_ref[...][None, :]
    seg = (lax.broadcasted_iota(jnp.int32, (N, G), 0) // NUM_KEYS
           == lax.broadcasted_iota(jnp.int32, (N, G), 1))
    segf = seg.astype(jnp.float32)
    gmax = jnp.max(jnp.where(seg, gate, -jnp.inf), axis=0, keepdims=True)
    gm_row = jnp.sum(jnp.where(seg, gmax, 0.0), axis=1, keepdims=True)
    ex = jnp.exp(gate - gm_row)
    den = lax.dot_general(segf, ex, (((0,), (0,)), ((), ())),
                          preferred_element_type=jnp.float32)
    den_row = jnp.dot(segf, den, preferred_element_type=jnp.float32)
    alpha = ex / jnp.maximum(den_row, 1e-16)
    gemb = lax.dot_general(segf, alpha * h, (((0,), (0,)), ((), ())),
                           preferred_element_type=jnp.float32)
    ph = jnp.dot(gemb, wp1_ref[...], preferred_element_type=jnp.float32) \
        + bp1_ref[...][None, :]
    mu = jnp.mean(ph, axis=-1, keepdims=True)
    var = jnp.mean((ph - mu) ** 2, axis=-1, keepdims=True)
    ph = lng_ref[...][None, :] * (ph - mu) / jnp.sqrt(var + 1e-5) \
        + lnb_ref[...][None, :]
    ph = jnp.maximum(ph, 0.0)
    pol_ref[...] = jnp.dot(ph, wp2_ref[...],
                           preferred_element_type=jnp.float32) \
        + bp2_ref[...][None, :]
    vh = jnp.maximum(jnp.dot(gemb, wv1_ref[...],
                             preferred_element_type=jnp.float32)
                     + bv1_ref[...][None, :], 0.0)
    val_ref[...] = jnp.tanh(jnp.dot(vh, wv2_ref[...],
                                    preferred_element_type=jnp.float32)
                            + bv2_ref[...][None, :])


def _pool_heads(h, p):
    return pl.pallas_call(
        _pool_heads_body,
        out_shape=(jax.ShapeDtypeStruct((G, 500), jnp.float32),
                   jax.ShapeDtypeStruct((G, 4), jnp.float32)),
    )(h, p["Wg1"], p["bg1"], p["Wg2"], p["bg2"],
      p["Wp1"], p["bp1"], p["ln_pg"], p["ln_pb"], p["Wp2"], p["bp2"],
      p["Wv1"], p["bv1"], p["Wv2"], p["bv2"])


K_EDGE = 48
N_WORKERS = 32
CHUNKS = -(-EF // (N_WORKERS * K_EDGE))
EP_TILE = CHUNKS * K_EDGE
EP = EP_TILE * N_WORKERS
ROWS_PER_SUB = NPAD // 16
DEN_PACK = 32
DEN_ROWS = NPAD // DEN_PACK
DROWS8 = DEN_ROWS // 8


def _butterfly_sum(u):
    def rot(x, k):
        idx = (lax.iota(jnp.int32, 16) + k) % 16
        return x + lax.gather(
            x, idx[:, None],
            lax.GatherDimensionNumbers(offset_dims=(),
                                       collapsed_slice_dims=(0,),
                                       start_index_map=(0,)),
            (1,), mode=lax.GatherScatterMode.PROMISE_IN_BOUNDS)
    for k in (8, 4, 2, 1):
        u = rot(u, k)
    return u


def _edge_sc_body(xl, xr, sdx, att, zeros_num,
                  out_num, out_den,
                  acc, accd, idx0, idx1, xl0, xl1, xr0, xr1,
                  dcmb0, dcmb1, att_v,
                  ga0, ga1, gb0, gb1, ns0, ns1, ds0, ds1):
    c = lax.axis_index("c")
    s = lax.axis_index("s")
    w = c * 16 + s
    idxb = (idx0, idx1)
    xlb = (xl0, xl1)
    xrb = (xr0, xr1)
    dcb = (dcmb0, dcmb1)
    gab = (ga0, ga1)
    gbb = (gb0, gb1)
    nsb = (ns0, ns1)
    dsb = (ds0, ds1)

    pltpu.sync_copy(zeros_num.at[pl.ds(s * ROWS_PER_SUB, ROWS_PER_SUB)],
                    acc.at[pl.ds(s * ROWS_PER_SUB, ROWS_PER_SUB)])

    @pl.when(s < 8)
    def _():
        pltpu.sync_copy(zeros_num.at[pl.ds(s * DROWS8, DROWS8)],
                        accd.at[pl.ds(s * DROWS8, DROWS8)])
    pltpu.sync_copy(att, att_v)

    plsc.subcore_barrier()

    def start_gathers(b, ci):
        pltpu.sync_copy(sdx.at[w, ci], idxb[b])
        pltpu.async_copy(xl.at[idxb[b].at[0]], xlb[b], gab[b])
        pltpu.async_copy(xr.at[idxb[b].at[1]], xrb[b], gbb[b])

    def wait_gathers(b):
        pltpu.make_async_copy(xl.at[pl.ds(0, K_EDGE)], xlb[b], gab[b]).wait()
        pltpu.make_async_copy(xr.at[pl.ds(0, K_EDGE)], xrb[b], gbb[b]).wait()

    def compute(b):
        xl_v = xlb[b]
        xr_v = xrb[b]
        dcmb_v = dcb[b]
        dst_r = idxb[b]

        def g_body(g, carry2):
            attv = [att_v[pl.ds(k * 16, 16)] for k in range(D // 16)]
            lane = lax.iota(jnp.int32, 16)
            head_mask = [lane == hh for hh in range(H)]
            zv = jnp.zeros((16,), jnp.float32)
            dstv = dst_r[1, pl.ds(g * 16, 16)]
            for tt in range(16):
                e = g * 16 + tt
                combo = zv
                for hh in range(H):
                    u = zv
                    xls = []
                    for jj in range(DH // 16):
                        off = hh * DH + jj * 16
                        xlv = xl_v[e, pl.ds(off, 16)]
                        xrv = xr_v[e, pl.ds(off, 16)]
                        sv = xlv + xrv
                        lr = jnp.where(sv > 0.0, sv, 0.2 * sv)
                        u = u + lr * attv[hh * 2 + jj]
                        xls.append(xlv)
                    exv = jnp.exp(_butterfly_sum(u))
                    for jj in range(DH // 16):
                        off = hh * DH + jj * 16
                        xl_v[e, pl.ds(off, 16)] = xls[jj] * exv
                    combo = jnp.where(head_mask[hh], exv, combo)
                q = dstv[tt] & (DEN_PACK - 1)
                off16 = (q & 3) * 4
                ridx = (lane - off16) & 15
                rot = lax.gather(
                    combo, ridx[:, None],
                    lax.GatherDimensionNumbers(offset_dims=(),
                                               collapsed_slice_dims=(0,),
                                               start_index_map=(0,)),
                    (1,), mode=lax.GatherScatterMode.PROMISE_IN_BOUNDS)
                for kk in range(8):
                    dcmb_v[e, pl.ds(kk * 16, 16)] = zv
                slot = lax.shift_right_logical(q, 2)
                dcmb_v[e, pl.ds(slot * 16, 16)] = rot
            return carry2
        lax.fori_loop(0, K_EDGE // 16, g_body, 0)

    def start_scatters(b):
        d1 = pltpu.async_copy(xlb[b], acc.at[idxb[b].at[1]], nsb[b],
                              add=True)
        d2 = pltpu.async_copy(dcb[b], accd.at[idxb[b].at[2]], dsb[b],
                              add=True)
        return d1, d2

    start_gathers(0, 0)

    def pair_body(it, carry):
        ci0 = it * 2
        start_gathers(1, ci0 + 1)
        wait_gathers(0)
        compute(0)
        d0a, d0b = start_scatters(0)
        wait_gathers(1)
        compute(1)
        d0a.wait()
        d0b.wait()
        d1a, d1b = start_scatters(1)

        @pl.when(it < CHUNKS // 2 - 1)
        def _():
            start_gathers(0, ci0 + 2)
        d1a.wait()
        d1b.wait()
        return carry

    lax.fori_loop(0, CHUNKS // 2, pair_body, 0)

    plsc.subcore_barrier()
    pltpu.sync_copy(acc.at[pl.ds(s * ROWS_PER_SUB, ROWS_PER_SUB)],
                    out_num.at[c, pl.ds(s * ROWS_PER_SUB, ROWS_PER_SUB)])

    @pl.when(s < 8)
    def _():
        pltpu.sync_copy(accd.at[pl.ds(s * DROWS8, DROWS8)],
                        out_den.at[c, pl.ds(s * DROWS8, DROWS8)])


def _edge_sc(xl_pad, xr_pad, sdx, att_flat, zeros_num):
    mesh = plsc.VectorSubcoreMesh(core_axis_name="c", subcore_axis_name="s")
    fn = pl.kernel(
        _edge_sc_body,
        out_type=(jax.ShapeDtypeStruct((2, NPAD, D), jnp.float32),
                  jax.ShapeDtypeStruct((2, DEN_ROWS, D), jnp.float32)),
        mesh=mesh,
        scratch_types=[
            pltpu.VMEM_SHARED((NPAD, D), jnp.float32),
            pltpu.VMEM_SHARED((DEN_ROWS, D), jnp.float32),
            pltpu.VMEM((3, K_EDGE), jnp.int32),
            pltpu.VMEM((3, K_EDGE), jnp.int32),
            pltpu.VMEM((K_EDGE, D), jnp.float32),
            pltpu.VMEM((K_EDGE, D), jnp.float32),
            pltpu.VMEM((K_EDGE, D), jnp.float32),
            pltpu.VMEM((K_EDGE, D), jnp.float32),
            pltpu.VMEM((K_EDGE, D), jnp.float32),
            pltpu.VMEM((K_EDGE, D), jnp.float32),
            pltpu.VMEM((D,), jnp.float32),
            pltpu.SemaphoreType.DMA,
            pltpu.SemaphoreType.DMA,
            pltpu.SemaphoreType.DMA,
            pltpu.SemaphoreType.DMA,
            pltpu.SemaphoreType.DMA,
            pltpu.SemaphoreType.DMA,
            pltpu.SemaphoreType.DMA,
            pltpu.SemaphoreType.DMA,
        ],
    )
    return fn(xl_pad, xr_pad, sdx, att_flat, zeros_num)


def _edge_stage(xl_pad, xr_pad, sdx, att, zeros_num):
    num, den = _edge_sc(xl_pad, xr_pad, sdx, att.reshape(-1), zeros_num)
    return num, den.reshape(2, NPAD, H)


def kernel(x, edge_index, batch_map, params):
    del batch_map
    loop = jnp.arange(N, dtype=edge_index.dtype)
    pad = jnp.full((EP - EF,), N, dtype=edge_index.dtype)
    src = jnp.concatenate([edge_index[0], loop, pad])
    dst = jnp.concatenate([edge_index[1], loop, pad])
    srcs = src.reshape(N_WORKERS, CHUNKS, K_EDGE)
    dsts = dst.reshape(N_WORKERS, CHUNKS, K_EDGE)
    sdx = jnp.stack([srcs, dsts, dsts // DEN_PACK], axis=2)
    zeros_num = jnp.zeros((NPAD, D), jnp.float32)
    pos_full = jnp.tile(params["pos"], (G, 1))
    h = _embed(x, params["W_emb"], params["b_emb"], pos_full)
    for i in range(L):
        xl, xr = _proj(h, params["Wl"][i], params["Wr"][i])
        num, den = _edge_stage(xl, xr, sdx, params["att"][i], zeros_num)
        h = _combine(num, den, params["bgat"][i], h,
                     params["ln_g"][i], params["ln_b"][i])
    return _pool_heads(h, params)


# speedup vs baseline: 1.2053x; 1.2053x over previous
import functools

import jax
import jax.numpy as jnp
from jax import lax
from jax.experimental import pallas as pl
from jax.experimental.pallas import tpu as pltpu
from jax.experimental.pallas import tpu_sc as plsc

N = 10000
E = 320000
G = 20
NUM_KEYS = 500
DIN = 16
D = 128
H = 4
DH = 32
L = 3

NPAD = 10240
EF = E + N


def _embed_body(x_ref, w_ref, b_ref, pos_ref, h_ref):
    h = jnp.maximum(jnp.dot(x_ref[...], w_ref[...],
                            preferred_element_type=jnp.float32)
                    + b_ref[...][None, :], 0.0)
    h_ref[...] = h + pos_ref[...]


def _embed(x, w, b, pos_full):
    return pl.pallas_call(
        _embed_body,
        out_shape=jax.ShapeDtypeStruct((N, D), jnp.float32),
    )(x, w, b, pos_full)


def _proj_body(h_ref, wl_ref, wr_ref, xl_ref, xr_ref):
    h = h_ref[...]
    xl_ref[:N, :] = jnp.dot(h, wl_ref[...], preferred_element_type=jnp.float32)
    xl_ref[N:, :] = jnp.zeros((NPAD - N, D), jnp.float32)
    xr_ref[:N, :] = jnp.dot(h, wr_ref[...], preferred_element_type=jnp.float32)
    xr_ref[N:, :] = jnp.zeros((NPAD - N, D), jnp.float32)


def _proj(h, wl, wr):
    return pl.pallas_call(
        _proj_body,
        out_shape=(jax.ShapeDtypeStruct((NPAD, D), jnp.float32),
                   jax.ShapeDtypeStruct((NPAD, D), jnp.float32)),
    )(h, wl, wr)


def _combine_body(num_ref, den_ref, b_ref, h_ref, lng_ref, lnb_ref, out_ref):
    num = num_ref[0, :N, :] + num_ref[1, :N, :]
    den = den_ref[0, :N, :H] + den_ref[1, :N, :H]
    den = jnp.maximum(den, 1e-16)
    r = 1.0 / den
    hsel = (lax.broadcasted_iota(jnp.int32, (H, D), 1) // DH
            == lax.broadcasted_iota(jnp.int32, (H, D), 0)).astype(jnp.float32)
    r128 = jnp.dot(r, hsel, preferred_element_type=jnp.float32)
    msg = jnp.maximum(num * r128 + b_ref[...][None, :], 0.0)
    hh = h_ref[...] + msg
    mu = jnp.mean(hh, axis=-1, keepdims=True)
    var = jnp.mean((hh - mu) ** 2, axis=-1, keepdims=True)
    out_ref[...] = (lng_ref[...][None, :] * (hh - mu)
                    / jnp.sqrt(var + 1e-5) + lnb_ref[...][None, :])


def _combine(num, den, bias, h, ln_g, ln_b):
    return pl.pallas_call(
        _combine_body,
        out_shape=jax.ShapeDtypeStruct((N, D), jnp.float32),
    )(num, den, bias, h, ln_g, ln_b)


def _pool_heads_body(h_ref, wg1_ref, bg1_ref, wg2_ref, bg2_ref,
                     wp1_ref, bp1_ref, lng_ref, lnb_ref, wp2_ref, bp2_ref,
                     wv1_ref, bv1_ref, wv2_ref, bv2_ref,
                     pol_ref, val_ref):
    h = h_ref[...]
    gate_hid = jnp.maximum(jnp.dot(h, wg1_ref[...],
                                   preferred_element_type=jnp.float32)
                           + bg1_ref[...][None, :], 0.0)
    gate = jnp.dot(gate_hid, wg2_ref[...],
                   preferred_element_type=jnp.float32) + bg2_ref[...][None, :]
    seg = (lax.broadcasted_iota(jnp.int32, (N, G), 0) // NUM_KEYS
           == lax.broadcasted_iota(jnp.int32, (N, G), 1))
    segf = seg.astype(jnp.float32)
    gmax = jnp.max(jnp.where(seg, gate, -jnp.inf), axis=0, keepdims=True)
    gm_row = jnp.sum(jnp.where(seg, gmax, 0.0), axis=1, keepdims=True)
    ex = jnp.exp(gate - gm_row)
    den = lax.dot_general(segf, ex, (((0,), (0,)), ((), ())),
                          preferred_element_type=jnp.float32)
    den_row = jnp.dot(segf, den, preferred_element_type=jnp.float32)
    alpha = ex / jnp.maximum(den_row, 1e-16)
    gemb = lax.dot_general(segf, alpha * h, (((0,), (0,)), ((), ())),
                           preferred_element_type=jnp.float32)
    ph = jnp.dot(gemb, wp1_ref[...], preferred_element_type=jnp.float32) \
        + bp1_ref[...][None, :]
    mu = jnp.mean(ph, axis=-1, keepdims=True)
    var = jnp.mean((ph - mu) ** 2, axis=-1, keepdims=True)
    ph = lng_ref[...][None, :] * (ph - mu) / jnp.sqrt(var + 1e-5) \
        + lnb_ref[...][None, :]
    ph = jnp.maximum(ph, 0.0)
    pol_ref[...] = jnp.dot(ph, wp2_ref[...],
                           preferred_element_type=jnp.float32) \
        + bp2_ref[...][None, :]
    vh = jnp.maximum(jnp.dot(gemb, wv1_ref[...],
                             preferred_element_type=jnp.float32)
                     + bv1_ref[...][None, :], 0.0)
    val_ref[...] = jnp.tanh(jnp.dot(vh, wv2_ref[...],
                                    preferred_element_type=jnp.float32)
                            + bv2_ref[...][None, :])


def _pool_heads(h, p):
    return pl.pallas_call(
        _pool_heads_body,
        out_shape=(jax.ShapeDtypeStruct((G, 500), jnp.float32),
                   jax.ShapeDtypeStruct((G, 4), jnp.float32)),
    )(h, p["Wg1"], p["bg1"], p["Wg2"], p["bg2"],
      p["Wp1"], p["bp1"], p["ln_pg"], p["ln_pb"], p["Wp2"], p["bp2"],
      p["Wv1"], p["bv1"], p["Wv2"], p["bv2"])


K_EDGE = 64
N_WORKERS = 32
CHUNKS = -(-EF // (N_WORKERS * K_EDGE))
EP_TILE = CHUNKS * K_EDGE
SUPER = 18
NSUP = CHUNKS // SUPER
EP = EP_TILE * N_WORKERS
ROWS_PER_SUB = NPAD // 16
DEN_PACK = 32
DEN_ROWS = NPAD // DEN_PACK
DROWS8 = DEN_ROWS // 8


def _butterfly_sum(u):
    def rot(x, k):
        idx = (lax.iota(jnp.int32, 16) + k) % 16
        return x + lax.gather(
            x, idx[:, None],
            lax.GatherDimensionNumbers(offset_dims=(),
                                       collapsed_slice_dims=(0,),
                                       start_index_map=(0,)),
            (1,), mode=lax.GatherScatterMode.PROMISE_IN_BOUNDS)
    for k in (8, 4, 2, 1):
        u = rot(u, k)
    return u


def _edge_sc_body(xl, xr, sdx, att, zeros_num,
                  out_num, out_den,
                  acc, accd, sup_v, xl0, xl1, xr0, xr1,
                  att_v,
                  ga0, ga1, gb0, gb1, ns0, ds0):
    c = lax.axis_index("c")
    s = lax.axis_index("s")
    w = c * 16 + s
    xlb = (xl0, xl1)
    xrb = (xr0, xr1)
    gab = (ga0, ga1)
    gbb = (gb0, gb1)

    pltpu.sync_copy(zeros_num.at[pl.ds(s * ROWS_PER_SUB, ROWS_PER_SUB)],
                    acc.at[pl.ds(s * ROWS_PER_SUB, ROWS_PER_SUB)])

    @pl.when(s < 8)
    def _():
        pltpu.sync_copy(zeros_num.at[pl.ds(s * DROWS8, DROWS8)],
                        accd.at[pl.ds(s * DROWS8, DROWS8)])
    pltpu.sync_copy(att, att_v)

    plsc.subcore_barrier()

    def start_gathers(b, j):
        pltpu.async_copy(xl.at[sup_v.at[j, 0]], xlb[b], gab[b])
        pltpu.async_copy(xr.at[sup_v.at[j, 1]], xrb[b], gbb[b])

    def wait_gathers(b):
        pltpu.make_async_copy(xl.at[pl.ds(0, K_EDGE)], xlb[b], gab[b]).wait()
        pltpu.make_async_copy(xr.at[pl.ds(0, K_EDGE)], xrb[b], gbb[b]).wait()

    def compute(b, j):
        xl_v = xlb[b]
        xr_v = xrb[b]

        def g_body(g, carry2):
            attv = [att_v[pl.ds(k * 16, 16)] for k in range(D // 16)]
            lane = lax.iota(jnp.int32, 16)
            head_mask = [lane == hh for hh in range(H)]
            zv = jnp.zeros((16,), jnp.float32)
            dstv = sup_v[j, 1, pl.ds(g * 16, 16)]
            for tt in range(16):
                e = g * 16 + tt
                combo = zv
                for hh in range(H):
                    u = zv
                    xls = []
                    for jj in range(DH // 16):
                        off = hh * DH + jj * 16
                        xlv = xl_v[e, pl.ds(off, 16)]
                        xrv = xr_v[e, pl.ds(off, 16)]
                        sv = xlv + xrv
                        lr = jnp.where(sv > 0.0, sv, 0.2 * sv)
                        u = u + lr * attv[hh * 2 + jj]
                        xls.append(xlv)
                    exv = jnp.exp(_butterfly_sum(u))
                    for jj in range(DH // 16):
                        off = hh * DH + jj * 16
                        xl_v[e, pl.ds(off, 16)] = xls[jj] * exv
                    combo = jnp.where(head_mask[hh], exv, combo)
                q = dstv[tt] & (DEN_PACK - 1)
                off16 = (q & 3) * 4
                ridx = (lane - off16) & 15
                rot = lax.gather(
                    combo, ridx[:, None],
                    lax.GatherDimensionNumbers(offset_dims=(),
                                               collapsed_slice_dims=(0,),
                                               start_index_map=(0,)),
                    (1,), mode=lax.GatherScatterMode.PROMISE_IN_BOUNDS)
                for kk in range(8):
                    xr_v[e, pl.ds(kk * 16, 16)] = zv
                slot = lax.shift_right_logical(q, 2)
                xr_v[e, pl.ds(slot * 16, 16)] = rot
            return carry2
        lax.fori_loop(0, K_EDGE // 16, g_body, 0)

    def start_scatters(b, j):
        d1 = pltpu.async_copy(xlb[b], acc.at[sup_v.at[j, 1]], ns0,
                              add=True)
        d2 = pltpu.async_copy(xrb[b], accd.at[sup_v.at[j, 2]], ds0,
                              add=True)
        return d1, d2

    def super_body(sp, carry):
        pltpu.sync_copy(sdx.at[w, pl.ds(sp * SUPER, SUPER)], sup_v)
        start_gathers(0, 0)

        def pair_body(it, carry2):
            j0 = it * 2
            start_gathers(1, j0 + 1)
            wait_gathers(0)
            compute(0, j0)
            d0a, d0b = start_scatters(0, j0)
            wait_gathers(1)
            compute(1, j0 + 1)
            d0a.wait()
            d0b.wait()
            d1a, d1b = start_scatters(1, j0 + 1)

            @pl.when(it < SUPER // 2 - 1)
            def _():
                start_gathers(0, j0 + 2)
            d1a.wait()
            d1b.wait()
            return carry2

        lax.fori_loop(0, SUPER // 2, pair_body, 0)
        return carry

    lax.fori_loop(0, NSUP, super_body, 0)

    plsc.subcore_barrier()
    pltpu.sync_copy(acc.at[pl.ds(s * ROWS_PER_SUB, ROWS_PER_SUB)],
                    out_num.at[c, pl.ds(s * ROWS_PER_SUB, ROWS_PER_SUB)])

    @pl.when(s < 8)
    def _():
        pltpu.sync_copy(accd.at[pl.ds(s * DROWS8, DROWS8)],
                        out_den.at[c, pl.ds(s * DROWS8, DROWS8)])


def _edge_sc(xl_pad, xr_pad, sdx, att_flat, zeros_num):
    mesh = plsc.VectorSubcoreMesh(core_axis_name="c", subcore_axis_name="s")
    fn = pl.kernel(
        _edge_sc_body,
        out_type=(jax.ShapeDtypeStruct((2, NPAD, D), jnp.float32),
                  jax.ShapeDtypeStruct((2, DEN_ROWS, D), jnp.float32)),
        mesh=mesh,
        scratch_types=[
            pltpu.VMEM_SHARED((NPAD, D), jnp.float32),
            pltpu.VMEM_SHARED((DEN_ROWS, D), jnp.float32),
            pltpu.VMEM((SUPER, 3, K_EDGE), jnp.int32),
            pltpu.VMEM((K_EDGE, D), jnp.float32),
            pltpu.VMEM((K_EDGE, D), jnp.float32),
            pltpu.VMEM((K_EDGE, D), jnp.float32),
            pltpu.VMEM((K_EDGE, D), jnp.float32),
            pltpu.VMEM((D,), jnp.float32),
            pltpu.SemaphoreType.DMA,
            pltpu.SemaphoreType.DMA,
            pltpu.SemaphoreType.DMA,
            pltpu.SemaphoreType.DMA,
            pltpu.SemaphoreType.DMA,
            pltpu.SemaphoreType.DMA,
        ],
    )
    return fn(xl_pad, xr_pad, sdx, att_flat, zeros_num)


def _edge_stage(xl_pad, xr_pad, sdx, att, zeros_num):
    num, den = _edge_sc(xl_pad, xr_pad, sdx, att.reshape(-1), zeros_num)
    return num, den.reshape(2, NPAD, H)


def kernel(x, edge_index, batch_map, params):
    del batch_map
    loop = jnp.arange(N, dtype=edge_index.dtype)
    pad = jnp.full((EP - EF,), N, dtype=edge_index.dtype)
    src = jnp.concatenate([edge_index[0], loop, pad])
    dst = jnp.concatenate([edge_index[1], loop, pad])
    srcs = src.reshape(N_WORKERS, CHUNKS, K_EDGE)
    dsts = dst.reshape(N_WORKERS, CHUNKS, K_EDGE)
    sdx = jnp.stack([srcs, dsts, dsts // DEN_PACK], axis=2)
    zeros_num = jnp.zeros((NPAD, D), jnp.float32)
    pos_full = jnp.tile(params["pos"], (G, 1))
    h = _embed(x, params["W_emb"], params["b_emb"], pos_full)
    for i in range(L):
        xl, xr = _proj(h, params["Wl"][i], params["Wr"][i])
        num, den = _edge_stage(xl, xr, sdx, params["att"][i], zeros_num)
        h = _combine(num, den, params["bgat"][i], h,
                     params["ln_g"][i], params["ln_b"][i])
    return _pool_heads(h, params)


# speedup vs baseline: 2.3091x; 1.9157x over previous
import functools

import jax
import jax.numpy as jnp
from jax import lax
from jax.experimental import pallas as pl
from jax.experimental.pallas import tpu as pltpu
from jax.experimental.pallas import tpu_sc as plsc

N = 10000
E = 320000
G = 20
NUM_KEYS = 500
DIN = 16
D = 128
H = 4
DH = 32
L = 3

NPAD = 10240
EF = E + N


def _embed_body(x_ref, w_ref, b_ref, pos_ref, h_ref):
    h = jnp.maximum(jnp.dot(x_ref[...], w_ref[...],
                            preferred_element_type=jnp.float32)
                    + b_ref[...][None, :], 0.0)
    h_ref[...] = h + pos_ref[...]


def _embed(x, w, b, pos_full):
    return pl.pallas_call(
        _embed_body,
        out_shape=jax.ShapeDtypeStruct((N, D), jnp.float32),
    )(x, w, b, pos_full)


def _proj_body(h_ref, wl_ref, wr_ref, xl_ref, xr_ref):
    h = h_ref[...]
    xl_ref[:N, :] = jnp.dot(h, wl_ref[...], preferred_element_type=jnp.float32)
    xl_ref[N:, :] = jnp.zeros((NPAD - N, D), jnp.float32)
    xr_ref[:N, :] = jnp.dot(h, wr_ref[...], preferred_element_type=jnp.float32)
    xr_ref[N:, :] = jnp.zeros((NPAD - N, D), jnp.float32)


def _proj(h, wl, wr):
    return pl.pallas_call(
        _proj_body,
        out_shape=(jax.ShapeDtypeStruct((NPAD, D), jnp.float32),
                   jax.ShapeDtypeStruct((NPAD, D), jnp.float32)),
    )(h, wl, wr)


def _combine_body(num_ref, den_ref, b_ref, h_ref, lng_ref, lnb_ref, out_ref):
    num = num_ref[0, :N, :] + num_ref[1, :N, :]
    den = den_ref[0, :N, :H] + den_ref[1, :N, :H]
    den = jnp.maximum(den, 1e-16)
    r = 1.0 / den
    hsel = (lax.broadcasted_iota(jnp.int32, (H, D), 1) // DH
            == lax.broadcasted_iota(jnp.int32, (H, D), 0)).astype(jnp.float32)
    r128 = jnp.dot(r, hsel, preferred_element_type=jnp.float32)
    msg = jnp.maximum(num * r128 + b_ref[...][None, :], 0.0)
    hh = h_ref[...] + msg
    mu = jnp.mean(hh, axis=-1, keepdims=True)
    var = jnp.mean((hh - mu) ** 2, axis=-1, keepdims=True)
    out_ref[...] = (lng_ref[...][None, :] * (hh - mu)
                    / jnp.sqrt(var + 1e-5) + lnb_ref[...][None, :])


def _combine(num, den, bias, h, ln_g, ln_b):
    return pl.pallas_call(
        _combine_body,
        out_shape=jax.ShapeDtypeStruct((N, D), jnp.float32),
    )(num, den, bias, h, ln_g, ln_b)


def _pool_heads_body(h_ref, wg1_ref, bg1_ref, wg2_ref, bg2_ref,
                     wp1_ref, bp1_ref, lng_ref, lnb_ref, wp2_ref, bp2_ref,
                     wv1_ref, bv1_ref, wv2_ref, bv2_ref,
                     pol_ref, val_ref):
    h = h_ref[...]
    gate_hid = jnp.maximum(jnp.dot(h, wg1_ref[...],
                                   preferred_element_type=jnp.float32)
                           + bg1_ref[...][None, :], 0.0)
    gate = jnp.dot(gate_hid, wg2_ref[...],
                   preferred_element_type=jnp.float32) + bg2_ref[...][None, :]
    seg = (lax.broadcasted_iota(jnp.int32, (N, G), 0) // NUM_KEYS
           == lax.broadcasted_iota(jnp.int32, (N, G), 1))
    segf = seg.astype(jnp.float32)
    gmax = jnp.max(jnp.where(seg, gate, -jnp.inf), axis=0, keepdims=True)
    gm_row = jnp.sum(jnp.where(seg, gmax, 0.0), axis=1, keepdims=True)
    ex = jnp.exp(gate - gm_row)
    den = lax.dot_general(segf, ex, (((0,), (0,)), ((), ())),
                          preferred_element_type=jnp.float32)
    den_row = jnp.dot(segf, den, preferred_element_type=jnp.float32)
    alpha = ex / jnp.maximum(den_row, 1e-16)
    gemb = lax.dot_general(segf, alpha * h, (((0,), (0,)), ((), ())),
                           preferred_element_type=jnp.float32)
    ph = jnp.dot(gemb, wp1_ref[...], preferred_element_type=jnp.float32) \
        + bp1_ref[...][None, :]
    mu = jnp.mean(ph, axis=-1, keepdims=True)
    var = jnp.mean((ph - mu) ** 2, axis=-1, keepdims=True)
    ph = lng_ref[...][None, :] * (ph - mu) / jnp.sqrt(var + 1e-5) \
        + lnb_ref[...][None, :]
    ph = jnp.maximum(ph, 0.0)
    pol_ref[...] = jnp.dot(ph, wp2_ref[...],
                           preferred_element_type=jnp.float32) \
        + bp2_ref[...][None, :]
    vh = jnp.maximum(jnp.dot(gemb, wv1_ref[...],
                             preferred_element_type=jnp.float32)
                     + bv1_ref[...][None, :], 0.0)
    val_ref[...] = jnp.tanh(jnp.dot(vh, wv2_ref[...],
                                    preferred_element_type=jnp.float32)
                            + bv2_ref[...][None, :])


def _pool_heads(h, p):
    return pl.pallas_call(
        _pool_heads_body,
        out_shape=(jax.ShapeDtypeStruct((G, 500), jnp.float32),
                   jax.ShapeDtypeStruct((G, 4), jnp.float32)),
    )(h, p["Wg1"], p["bg1"], p["Wg2"], p["bg2"],
      p["Wp1"], p["bp1"], p["ln_pg"], p["ln_pb"], p["Wp2"], p["bp2"],
      p["Wv1"], p["bv1"], p["Wv2"], p["bv2"])


K_EDGE = 64
N_WORKERS = 32
CHUNKS = -(-EF // (N_WORKERS * K_EDGE))
EP_TILE = CHUNKS * K_EDGE
SUPER = 18
NSUP = CHUNKS // SUPER
EP = EP_TILE * N_WORKERS
ROWS_PER_SUB = NPAD // 16
DEN_PACK = 32
DEN_ROWS = NPAD // DEN_PACK
DROWS8 = DEN_ROWS // 8


def _butterfly_sum(u):
    def rot(x, k):
        idx = (lax.iota(jnp.int32, 16) + k) % 16
        return x + lax.gather(
            x, idx[:, None],
            lax.GatherDimensionNumbers(offset_dims=(),
                                       collapsed_slice_dims=(0,),
                                       start_index_map=(0,)),
            (1,), mode=lax.GatherScatterMode.PROMISE_IN_BOUNDS)
    for k in (8, 4, 2, 1):
        u = rot(u, k)
    return u


def _edge_sc_body(xl, xr, sdx, att, zeros_num,
                  out_num, out_den,
                  acc, accd, sup_v, xl0, xl1, xr0, xr1,
                  att_v,
                  ga0, ga1, gb0, gb1, ns0, ds0):
    c = lax.axis_index("c")
    s = lax.axis_index("s")
    w = c * 16 + s
    xlb = (xl0, xl1)
    xrb = (xr0, xr1)
    gab = (ga0, ga1)
    gbb = (gb0, gb1)

    pltpu.sync_copy(zeros_num.at[pl.ds(s * ROWS_PER_SUB, ROWS_PER_SUB)],
                    acc.at[pl.ds(s * ROWS_PER_SUB, ROWS_PER_SUB)])

    @pl.when(s < 8)
    def _():
        pltpu.sync_copy(zeros_num.at[pl.ds(s * DROWS8, DROWS8)],
                        accd.at[pl.ds(s * DROWS8, DROWS8)])
    pltpu.sync_copy(att, att_v)

    plsc.subcore_barrier()

    def start_gathers(b, j):
        pltpu.async_copy(xl.at[sup_v.at[j, 0]], xlb[b], gab[b])
        pltpu.async_copy(xr.at[sup_v.at[j, 1]], xrb[b], gbb[b])

    def wait_gathers(b):
        pltpu.make_async_copy(xl.at[pl.ds(0, K_EDGE)], xlb[b], gab[b]).wait()
        pltpu.make_async_copy(xr.at[pl.ds(0, K_EDGE)], xrb[b], gbb[b]).wait()

    def compute(b, j):
        xl_v = xlb[b]
        xr_v = xrb[b]

        @functools.partial(plsc.parallel_loop, 0, K_EDGE // 16)
        def g_body(g):
            attv = [att_v[pl.ds(k * 16, 16)] for k in range(D // 16)]
            lane = lax.iota(jnp.int32, 16)
            head_mask = [lane == hh for hh in range(H)]
            zv = jnp.zeros((16,), jnp.float32)
            dstv = sup_v[j, 1, pl.ds(g * 16, 16)]
            for tt in range(16):
                e = g * 16 + tt
                combo = zv
                for hh in range(H):
                    u = zv
                    xls = []
                    for jj in range(DH // 16):
                        off = hh * DH + jj * 16
                        xlv = xl_v[e, pl.ds(off, 16)]
                        xrv = xr_v[e, pl.ds(off, 16)]
                        sv = xlv + xrv
                        lr = jnp.maximum(sv, 0.2 * sv)
                        u = u + lr * attv[hh * 2 + jj]
                        xls.append(xlv)
                    exv = jnp.exp(_butterfly_sum(u))
                    for jj in range(DH // 16):
                        off = hh * DH + jj * 16
                        xl_v[e, pl.ds(off, 16)] = xls[jj] * exv
                    combo = jnp.where(head_mask[hh], exv, combo)
                q = dstv[tt] & (DEN_PACK - 1)
                off16 = (q & 3) * 4
                ridx = (lane - off16) & 15
                rot = lax.gather(
                    combo, ridx[:, None],
                    lax.GatherDimensionNumbers(offset_dims=(),
                                               collapsed_slice_dims=(0,),
                                               start_index_map=(0,)),
                    (1,), mode=lax.GatherScatterMode.PROMISE_IN_BOUNDS)
                for kk in range(8):
                    xr_v[e, pl.ds(kk * 16, 16)] = zv
                slot = lax.shift_right_logical(q, 2)
                xr_v[e, pl.ds(slot * 16, 16)] = rot

    def start_scatters(b, j):
        d1 = pltpu.async_copy(xlb[b], acc.at[sup_v.at[j, 1]], ns0,
                              add=True)
        d2 = pltpu.async_copy(xrb[b], accd.at[sup_v.at[j, 2]], ds0,
                              add=True)
        return d1, d2

    def super_body(sp, carry):
        pltpu.sync_copy(sdx.at[w, pl.ds(sp * SUPER, SUPER)], sup_v)
        start_gathers(0, 0)

        def pair_body(it, carry2):
            j0 = it * 2
            start_gathers(1, j0 + 1)
            wait_gathers(0)
            compute(0, j0)
            d0a, d0b = start_scatters(0, j0)
            wait_gathers(1)
            compute(1, j0 + 1)
            d0a.wait()
            d0b.wait()
            d1a, d1b = start_scatters(1, j0 + 1)

            @pl.when(it < SUPER // 2 - 1)
            def _():
                start_gathers(0, j0 + 2)
            d1a.wait()
            d1b.wait()
            return carry2

        lax.fori_loop(0, SUPER // 2, pair_body, 0)
        return carry

    lax.fori_loop(0, NSUP, super_body, 0)

    plsc.subcore_barrier()
    pltpu.sync_copy(acc.at[pl.ds(s * ROWS_PER_SUB, ROWS_PER_SUB)],
                    out_num.at[c, pl.ds(s * ROWS_PER_SUB, ROWS_PER_SUB)])

    @pl.when(s < 8)
    def _():
        pltpu.sync_copy(accd.at[pl.ds(s * DROWS8, DROWS8)],
                        out_den.at[c, pl.ds(s * DROWS8, DROWS8)])


def _edge_sc(xl_pad, xr_pad, sdx, att_flat, zeros_num):
    mesh = plsc.VectorSubcoreMesh(core_axis_name="c", subcore_axis_name="s")
    fn = pl.kernel(
        _edge_sc_body,
        out_type=(jax.ShapeDtypeStruct((2, NPAD, D), jnp.float32),
                  jax.ShapeDtypeStruct((2, DEN_ROWS, D), jnp.float32)),
        mesh=mesh,
        scratch_types=[
            pltpu.VMEM_SHARED((NPAD, D), jnp.float32),
            pltpu.VMEM_SHARED((DEN_ROWS, D), jnp.float32),
            pltpu.VMEM((SUPER, 3, K_EDGE), jnp.int32),
            pltpu.VMEM((K_EDGE, D), jnp.float32),
            pltpu.VMEM((K_EDGE, D), jnp.float32),
            pltpu.VMEM((K_EDGE, D), jnp.float32),
            pltpu.VMEM((K_EDGE, D), jnp.float32),
            pltpu.VMEM((D,), jnp.float32),
            pltpu.SemaphoreType.DMA,
            pltpu.SemaphoreType.DMA,
            pltpu.SemaphoreType.DMA,
            pltpu.SemaphoreType.DMA,
            pltpu.SemaphoreType.DMA,
            pltpu.SemaphoreType.DMA,
        ],
    )
    return fn(xl_pad, xr_pad, sdx, att_flat, zeros_num)


def _edge_stage(xl_pad, xr_pad, sdx, att, zeros_num):
    num, den = _edge_sc(xl_pad, xr_pad, sdx, att.reshape(-1), zeros_num)
    return num, den.reshape(2, NPAD, H)


def kernel(x, edge_index, batch_map, params):
    del batch_map
    loop = jnp.arange(N, dtype=edge_index.dtype)
    pad = jnp.full((EP - EF,), N, dtype=edge_index.dtype)
    src = jnp.concatenate([edge_index[0], loop, pad])
    dst = jnp.concatenate([edge_index[1], loop, pad])
    srcs = src.reshape(N_WORKERS, CHUNKS, K_EDGE)
    dsts = dst.reshape(N_WORKERS, CHUNKS, K_EDGE)
    sdx = jnp.stack([srcs, dsts, dsts // DEN_PACK], axis=2)
    zeros_num = jnp.zeros((NPAD, D), jnp.float32)
    pos_full = jnp.tile(params["pos"], (G, 1))
    h = _embed(x, params["W_emb"], params["b_emb"], pos_full)
    for i in range(L):
        xl, xr = _proj(h, params["Wl"][i], params["Wr"][i])
        num, den = _edge_stage(xl, xr, sdx, params["att"][i], zeros_num)
        h = _combine(num, den, params["bgat"][i], h,
                     params["ln_g"][i], params["ln_b"][i])
    return _pool_heads(h, params)
